# baseline (speedup 1.0000x reference)
import functools

import jax
import jax.numpy as jnp
from jax.experimental import pallas as pl
from jax.experimental.pallas import tpu as pltpu

_SMOOTHING = 0.1
_CONFIDENCE = 1.0 - _SMOOTHING


def _row_pass_body(x_ref, t_ref, o_ref, *, num_classes, ch=128):
    r = x_ref.shape[0]
    c = x_ref.shape[1]
    nch = c // ch
    t = t_ref[...]
    lane = jax.lax.broadcasted_iota(jnp.int32, (r, ch), 1)

    def p1(i, acc):
        mx, sx, xt = acc
        xc = x_ref[:, pl.ds(i * ch, ch)]
        mx = jnp.maximum(mx, xc)
        sx = sx + xc
        xt = xt + jnp.where(lane == t - i * ch, xc, 0.0)
        return mx, sx, xt

    mx, sx, xt = jax.lax.fori_loop(
        0, nch, p1,
        (jnp.full((r, ch), -jnp.inf, jnp.float32),
         jnp.zeros((r, ch), jnp.float32),
         jnp.zeros((r, ch), jnp.float32)))
    bm = jnp.max(mx, axis=1, keepdims=True)
    sxr = jnp.sum(sx, axis=1, keepdims=True)
    xtr = jnp.sum(xt, axis=1, keepdims=True)

    def p2(i, s):
        xc = x_ref[:, pl.ds(i * ch, ch)]
        return s + jnp.exp(xc - bm)

    s = jax.lax.fori_loop(0, nch, p2, jnp.zeros((r, ch), jnp.float32))
    sr = jnp.sum(s, axis=1, keepdims=True)

    eps = _SMOOTHING / (num_classes - 1)
    lse = bm + jnp.log(sr)
    o_ref[...] = (eps * (num_classes * lse - sxr)
                  + (_CONFIDENCE - eps) * (lse - xtr))


def _mean_body(r_ref, o_ref):
    n = r_ref.shape[0]
    o_ref[...] = jnp.sum(r_ref[...], keepdims=True) * (1.0 / n)


def kernel(outputs, targets):
    n, c = outputs.shape
    r = 128 if n % 128 == 0 else n
    t2 = targets.reshape(n, 1)

    row_losses = pl.pallas_call(
        functools.partial(_row_pass_body, num_classes=c),
        grid=(n // r,),
        in_specs=[
            pl.BlockSpec((r, c), lambda i: (i, 0)),
            pl.BlockSpec((r, 1), lambda i: (i, 0)),
        ],
        out_specs=pl.BlockSpec((r, 1), lambda i: (i, 0)),
        out_shape=jax.ShapeDtypeStruct((n, 1), jnp.float32),
        compiler_params=pltpu.CompilerParams(
            dimension_semantics=("arbitrary",),
        ),
    )(outputs, t2)

    loss = pl.pallas_call(
        _mean_body,
        out_shape=jax.ShapeDtypeStruct((1, 1), jnp.float32),
    )(row_losses)
    return loss[0, 0]


# speedup vs baseline: 4.8152x; 4.8152x over previous
import functools

import jax
import jax.numpy as jnp
from jax import lax
from jax.experimental import pallas as pl
from jax.experimental.pallas import tpu as pltpu

_SMOOTHING = 0.1
_CONFIDENCE = 1.0 - _SMOOTHING


def _row_pass_body(x_ref, t_ref, o_ref, o2_ref, *, num_classes):
    x = x_ref[...]
    r = x.shape[0]
    bm = jnp.max(x, axis=1, keepdims=True)
    s = jnp.sum(jnp.exp(x - bm), axis=1, keepdims=True)
    sx = jnp.sum(x, axis=1, keepdims=True)
    eps = _SMOOTHING / (num_classes - 1)
    lse = bm + jnp.log(s)
    o_ref[...] = (eps * (num_classes * lse - sx)
                  + (_CONFIDENCE - eps) * lse)

    lane_io = lax.broadcasted_iota(jnp.int32, (8, 128), 1)
    sub_io = lax.broadcasted_iota(jnp.int32, (8, 128), 0)
    acc = jnp.zeros((8, 128), jnp.float32)
    for rr in range(r):
        t_s = t_ref[rr, 0]
        toff = (t_s // 128) * 128
        g8 = (rr // 8) * 8
        blk = x_ref[pl.ds(g8, 8), pl.ds(toff, 128)]
        hit = (lane_io == t_s - toff) & (sub_io == rr - g8)
        acc = acc + jnp.where(hit, blk, 0.0)
    o2_ref[...] = jnp.sum(acc).reshape(1, 1, 1)


def _mean_body(r_ref, xt_ref, o_ref, *, num_classes, n):
    eps = _SMOOTHING / (num_classes - 1)
    o_ref[...] = (jnp.sum(r_ref[...], keepdims=True)
                  - (_CONFIDENCE - eps) * jnp.sum(xt_ref[...], keepdims=True)
                  ) * (1.0 / n)


def kernel(outputs, targets):
    n, c = outputs.shape
    r = 128 if n % 128 == 0 else n
    nb = n // r
    t2 = targets.reshape(n, 1)

    row_losses, xt_part = pl.pallas_call(
        functools.partial(_row_pass_body, num_classes=c),
        grid=(nb,),
        in_specs=[
            pl.BlockSpec((r, c), lambda i: (i, 0)),
            pl.BlockSpec((r, 1), lambda i: (i, 0), memory_space=pltpu.SMEM),
        ],
        out_specs=[
            pl.BlockSpec((r, 1), lambda i: (i, 0)),
            pl.BlockSpec((1, 1, 1), lambda i: (i, 0, 0)),
        ],
        out_shape=[
            jax.ShapeDtypeStruct((n, 1), jnp.float32),
            jax.ShapeDtypeStruct((nb, 1, 1), jnp.float32),
        ],
        compiler_params=pltpu.CompilerParams(
            dimension_semantics=("arbitrary",),
        ),
    )(outputs, t2)

    loss = pl.pallas_call(
        functools.partial(_mean_body, num_classes=c, n=n),
        out_shape=jax.ShapeDtypeStruct((1, 1), jnp.float32),
    )(row_losses, xt_part.reshape(nb, 1))
    return loss[0, 0]


# speedup vs baseline: 5.2257x; 1.0853x over previous
import functools

import jax
import jax.numpy as jnp
from jax import lax
from jax.experimental import pallas as pl
from jax.experimental.pallas import tpu as pltpu

_SMOOTHING = 0.1
_CONFIDENCE = 1.0 - _SMOOTHING


def _row_pass_body(x_ref, t_ref, o_ref, o2_ref, *, num_classes):
    r, c = x_ref.shape
    ch = 128
    nch = c // ch

    mx = x_ref[:, pl.ds(0, ch)]
    sxa = x_ref[:, pl.ds(0, ch)]
    for k in range(1, nch):
        xc = x_ref[:, pl.ds(k * ch, ch)]
        mx = jnp.maximum(mx, xc)
        sxa = sxa + xc
    bm = jnp.max(mx, axis=1, keepdims=True)
    sx = jnp.sum(sxa, axis=1, keepdims=True)

    sa = jnp.zeros((r, ch), jnp.float32)
    for k in range(nch):
        sa = sa + jnp.exp(x_ref[:, pl.ds(k * ch, ch)] - bm)
    s = jnp.sum(sa, axis=1, keepdims=True)

    eps = _SMOOTHING / (num_classes - 1)
    lse = bm + jnp.log(s)
    o_ref[...] = (eps * (num_classes * lse - sx)
                  + (_CONFIDENCE - eps) * lse)

    lane_io = lax.broadcasted_iota(jnp.int32, (8, 128), 1)
    sub_io = lax.broadcasted_iota(jnp.int32, (8, 128), 0)
    acc = jnp.zeros((8, 128), jnp.float32)
    for rr in range(r):
        t_s = t_ref[rr, 0]
        toff = (t_s // 128) * 128
        g8 = (rr // 8) * 8
        blk = x_ref[pl.ds(g8, 8), pl.ds(toff, 128)]
        hit = (lane_io == t_s - toff) & (sub_io == rr - g8)
        acc = acc + jnp.where(hit, blk, 0.0)
    o2_ref[...] = jnp.sum(acc).reshape(1, 1, 1)


def _mean_body(r_ref, xt_ref, o_ref, *, num_classes, n):
    eps = _SMOOTHING / (num_classes - 1)
    o_ref[...] = (jnp.sum(r_ref[...], keepdims=True)
                  - (_CONFIDENCE - eps) * jnp.sum(xt_ref[...], keepdims=True)
                  ) * (1.0 / n)


def kernel(outputs, targets):
    n, c = outputs.shape
    r = 128 if n % 128 == 0 else n
    nb = n // r
    t2 = targets.reshape(n, 1)

    row_losses, xt_part = pl.pallas_call(
        functools.partial(_row_pass_body, num_classes=c),
        grid=(nb,),
        in_specs=[
            pl.BlockSpec((r, c), lambda i: (i, 0)),
            pl.BlockSpec((r, 1), lambda i: (i, 0), memory_space=pltpu.SMEM),
        ],
        out_specs=[
            pl.BlockSpec((r, 1), lambda i: (i, 0)),
            pl.BlockSpec((1, 1, 1), lambda i: (i, 0, 0)),
        ],
        out_shape=[
            jax.ShapeDtypeStruct((n, 1), jnp.float32),
            jax.ShapeDtypeStruct((nb, 1, 1), jnp.float32),
        ],
        compiler_params=pltpu.CompilerParams(
            dimension_semantics=("arbitrary",),
        ),
    )(outputs, t2)

    loss = pl.pallas_call(
        functools.partial(_mean_body, num_classes=c, n=n),
        out_shape=jax.ShapeDtypeStruct((1, 1), jnp.float32),
    )(row_losses, xt_part.reshape(nb, 1))
    return loss[0, 0]
